# Initial kernel scaffold; baseline (speedup 1.0000x reference)
import jax
import jax.numpy as jnp
from jax.experimental import pallas as pl


def kernel(node_list, edge_list, edge_att, W_l, b_l, W_r, b_r, att, W_e, bias, lin_W, lin_b):
    raise NotImplementedError("write your pallas kernel here")


# speedup vs baseline: 1.9765x; 1.9765x over previous
import functools

import jax
import jax.numpy as jnp
from jax import lax
from jax.experimental import pallas as pl
from jax.experimental.pallas import tpu as pltpu
from jax.experimental.pallas import tpu_sc as plsc

N_NODES = 10000
N_EDGES = 320000
FEAT = 128
NC = 2
NS = 16
NW = NC * NS
EPW = N_EDGES // NW
CH = 80
NCHUNK = EPW // CH
NG = CH // 16

_mesh = plsc.VectorSubcoreMesh(core_axis_name="c", subcore_axis_name="s")


def _proj_body(x_ref, wlT_ref, wrT_ref, bl_ref, br_ref, xl_ref, xr_ref):
    x = x_ref[...]
    xl_ref[...] = jnp.dot(x, wlT_ref[...], preferred_element_type=jnp.float32) + bl_ref[...]
    xr_ref[...] = jnp.dot(x, wrT_ref[...], preferred_element_type=jnp.float32) + br_ref[...]


def _proj(x, wlT, wrT, bl, br):
    blk = 1000
    return pl.pallas_call(
        _proj_body,
        grid=(N_NODES // blk,),
        in_specs=[
            pl.BlockSpec((blk, FEAT), lambda i: (i, 0)),
            pl.BlockSpec((FEAT, FEAT), lambda i: (0, 0)),
            pl.BlockSpec((FEAT, FEAT), lambda i: (0, 0)),
            pl.BlockSpec((1, FEAT), lambda i: (0, 0)),
            pl.BlockSpec((1, FEAT), lambda i: (0, 0)),
        ],
        out_specs=[
            pl.BlockSpec((blk, FEAT), lambda i: (i, 0)),
            pl.BlockSpec((blk, FEAT), lambda i: (i, 0)),
        ],
        out_shape=[
            jax.ShapeDtypeStruct((N_NODES, FEAT), jnp.float32),
            jax.ShapeDtypeStruct((N_NODES, FEAT), jnp.float32),
        ],
    )(x, wlT, wrT, bl, br)


@functools.partial(
    pl.kernel,
    out_type=(
        jax.ShapeDtypeStruct((N_EDGES,), jnp.float32),
        jax.ShapeDtypeStruct((NW, 16), jnp.float32),
    ),
    mesh=_mesh,
    compiler_params=pltpu.CompilerParams(needs_layout_passes=False),
    scratch_types=[
        pltpu.VMEM((CH,), jnp.int32),
        pltpu.VMEM((CH,), jnp.int32),
        pltpu.VMEM((CH,), jnp.float32),
        pltpu.VMEM((CH, FEAT), jnp.float32),
        pltpu.VMEM((CH, FEAT), jnp.float32),
        pltpu.VMEM((CH,), jnp.float32),
        pltpu.VMEM((FEAT, 16), jnp.float32),
        pltpu.VMEM((FEAT, 16), jnp.float32),
        pltpu.VMEM((16,), jnp.float32),
        pltpu.SemaphoreType.DMA,
        pltpu.SemaphoreType.DMA,
    ],
)
def _edge_logits(xl_hbm, xr_hbm, src_hbm, dst_hbm, ea_hbm, att_hbm, we_hbm,
                 alpha_hbm, wmax_hbm,
                 srcv, dstv, eav, xlr, xrr, alphabuf, attv, wev, maxbuf,
                 sem1, sem2):
    wid = lax.axis_index("s") * NC + lax.axis_index("c")
    base = wid * EPW
    pltpu.sync_copy(att_hbm, attv)
    pltpu.sync_copy(we_hbm, wev)
    rows = [lax.iota(jnp.int32, 16) + (g * 16) for g in range(NG)]

    def chunk_body(k, carry_max):
        off = base + k * CH
        pltpu.sync_copy(src_hbm.at[pl.ds(off, CH)], srcv)
        pltpu.sync_copy(dst_hbm.at[pl.ds(off, CH)], dstv)
        pltpu.sync_copy(ea_hbm.at[pl.ds(off, CH)], eav)
        cl = pltpu.async_copy(xl_hbm.at[srcv], xlr, sem1)
        cr = pltpu.async_copy(xr_hbm.at[dstv], xrr, sem2)
        cl.wait()
        cr.wait()
        eags = [eav[pl.ds(g * 16, 16)] for g in range(NG)]

        def feat_body(ci, accs):
            col = jnp.full((16,), ci, jnp.int32)
            a_c = attv[ci]
            w_c = wev[ci]
            outs = []
            for g in range(NG):
                xl16 = plsc.load_gather(xlr, [rows[g], col])
                xr16 = plsc.load_gather(xrr, [rows[g], col])
                m = xl16 + xr16 + w_c * eags[g]
                m = jnp.maximum(m, 0.2 * m)
                outs.append(accs[g] + a_c * m)
            return tuple(outs)

        accs = lax.fori_loop(0, FEAT, feat_body,
                             tuple(jnp.zeros((16,), jnp.float32) for _ in range(NG)))
        newmax = carry_max
        for g in range(NG):
            alphabuf[pl.ds(g * 16, 16)] = accs[g]
            newmax = jnp.maximum(newmax, accs[g])
        pltpu.sync_copy(alphabuf, alpha_hbm.at[pl.ds(off, CH)])
        return newmax

    mx = lax.fori_loop(0, NCHUNK, chunk_body,
                       jnp.full((16,), -3.0e38, jnp.float32))
    maxbuf[...] = mx
    pltpu.sync_copy(maxbuf, wmax_hbm.at[wid])


@functools.partial(
    pl.kernel,
    out_type=(
        jax.ShapeDtypeStruct((N_EDGES,), jnp.float32),
        jax.ShapeDtypeStruct((NW * N_NODES,), jnp.float32),
    ),
    mesh=_mesh,
    compiler_params=pltpu.CompilerParams(needs_layout_passes=False),
    scratch_types=[
        pltpu.VMEM((EPW,), jnp.float32),
        pltpu.VMEM((EPW,), jnp.int32),
        pltpu.VMEM((N_NODES,), jnp.float32),
        pltpu.VMEM((NW, 16), jnp.float32),
    ],
)
def _edge_exp(alpha_hbm, wmax_hbm, dst_hbm,
              ex_hbm, ssump_hbm,
              alphav, dstv, ssuml, wmaxv):
    wid = lax.axis_index("s") * NC + lax.axis_index("c")
    base = wid * EPW
    pltpu.sync_copy(wmax_hbm, wmaxv)
    m = wmaxv[0]
    for i in range(1, NW):
        m = jnp.maximum(m, wmaxv[i])
    g_shift = jnp.max(m)
    pltpu.sync_copy(alpha_hbm.at[pl.ds(base, EPW)], alphav)
    pltpu.sync_copy(dst_hbm.at[pl.ds(base, EPW)], dstv)

    def zero_body(i, _):
        ssuml[pl.ds(i * 16, 16)] = jnp.zeros((16,), jnp.float32)
        return 0

    lax.fori_loop(0, N_NODES // 16, zero_body, 0)

    def body(i, _):
        a = alphav[pl.ds(i * 16, 16)]
        e = jnp.exp(a - g_shift)
        alphav[pl.ds(i * 16, 16)] = e
        d = dstv[pl.ds(i * 16, 16)]
        plsc.addupdate_scatter(ssuml, [d], e)
        return 0

    lax.fori_loop(0, EPW // 16, body, 0)
    pltpu.sync_copy(alphav, ex_hbm.at[pl.ds(base, EPW)])
    pltpu.sync_copy(ssuml, ssump_hbm.at[pl.ds(wid * N_NODES, N_NODES)])


@functools.partial(
    pl.kernel,
    out_type=(
        jax.ShapeDtypeStruct((N_EDGES,), jnp.float32),
        jax.ShapeDtypeStruct((NC, N_NODES, FEAT), jnp.float32),
    ),
    mesh=_mesh,
    compiler_params=pltpu.CompilerParams(needs_layout_passes=False),
    scratch_types=[
        pltpu.VMEM((N_NODES,), jnp.float32),
        pltpu.VMEM((N_NODES,), jnp.float32),
        pltpu.VMEM((CH,), jnp.int32),
        pltpu.VMEM((1, CH), jnp.int32),
        pltpu.VMEM((CH,), jnp.float32),
        pltpu.VMEM((CH,), jnp.float32),
        pltpu.VMEM((CH, FEAT), jnp.float32),
        pltpu.VMEM((CH, FEAT), jnp.float32),
        pltpu.VMEM_SHARED((N_NODES, FEAT), jnp.float32),
        pltpu.SemaphoreType.DMA,
    ],
)
def _edge_norm(ssump_hbm, ex_hbm, src_hbm, dst_hbm, xl_hbm,
               alphan_hbm, outp_hbm,
               ssumv, tmpv, srcv, dstv, exv, anbuf, xlr, zbuf,
               out_sh, sem):
    cid = lax.axis_index("c")
    sid = lax.axis_index("s")
    wid = sid * NC + cid
    base = wid * EPW
    rows = [lax.iota(jnp.int32, 16) + (g * 16) for g in range(NG)]

    pltpu.sync_copy(ssump_hbm.at[pl.ds(0, N_NODES)], ssumv)
    for r in range(1, NW):
        pltpu.sync_copy(ssump_hbm.at[pl.ds(r * N_NODES, N_NODES)], tmpv)

        def add_body(j, _):
            ssumv[pl.ds(j * 16, 16)] = (ssumv[pl.ds(j * 16, 16)]
                                        + tmpv[pl.ds(j * 16, 16)])
            return 0

        lax.fori_loop(0, N_NODES // 16, add_body, 0)

    @pl.when(sid < 10)
    def _():
        def zero_body(i, _):
            r = i // (FEAT // 16)
            c = i % (FEAT // 16)
            zbuf[r, pl.ds(c * 16, 16)] = jnp.zeros((16,), jnp.float32)
            return 0

        lax.fori_loop(0, CH * (FEAT // 16), zero_body, 0)
        for t in range(12):
            pltpu.sync_copy(zbuf, out_sh.at[pl.ds(sid * 1000 + t * CH, CH)])
        pltpu.sync_copy(zbuf.at[pl.ds(0, 40)],
                        out_sh.at[pl.ds(sid * 1000 + 960, 40)])

    plsc.subcore_barrier()

    def chunk_body(k, _):
        off = base + k * CH
        pltpu.sync_copy(src_hbm.at[pl.ds(off, CH)], srcv)
        pltpu.sync_copy(dst_hbm.at[pl.ds(off, CH)], dstv.at[0])
        pltpu.sync_copy(ex_hbm.at[pl.ds(off, CH)], exv)
        pltpu.async_copy(xl_hbm.at[srcv], xlr, sem).wait()
        angs = []
        for g in range(NG):
            d16 = dstv[0, pl.ds(g * 16, 16)]
            s16 = plsc.load_gather(ssumv, [d16])
            an = exv[pl.ds(g * 16, 16)] / (s16 + 1e-16)
            anbuf[pl.ds(g * 16, 16)] = an
            angs.append(an)

        def feat_body(ci, _):
            col = jnp.full((16,), ci, jnp.int32)
            for g in range(NG):
                v = plsc.load_gather(xlr, [rows[g], col])
                plsc.store_scatter(xlr, [rows[g], col], v * angs[g])
            return 0

        lax.fori_loop(0, FEAT, feat_body, 0)
        pltpu.sync_copy(anbuf, alphan_hbm.at[pl.ds(off, CH)])
        pltpu.sync_copy(xlr, out_sh.at[dstv.at[0]], add=True)
        return 0

    lax.fori_loop(0, NCHUNK, chunk_body, 0)
    plsc.subcore_barrier()

    @pl.when(sid < 10)
    def _():
        pltpu.sync_copy(out_sh.at[pl.ds(sid * 1000, 1000)],
                        outp_hbm.at[cid].at[pl.ds(sid * 1000, 1000)])


def _head_body(p0_ref, p1_ref, b_ref, lw_ref, lb_ref, out_ref):
    h = jnp.maximum(p0_ref[...] + p1_ref[...] + b_ref[...], 0.0)
    pooled = jnp.sum(h, axis=0) * (1.0 / N_NODES)
    s0 = jnp.sum(lw_ref[0, :] * pooled) + lb_ref[0, 0]
    s1 = jnp.sum(lw_ref[1, :] * pooled) + lb_ref[0, 1]
    m = jnp.maximum(s0, s1)
    e0 = jnp.exp(s0 - m)
    e1 = jnp.exp(s1 - m)
    out_ref[0, 0] = e0 / (e0 + e1)
    out_ref[0, 1] = e1 / (e0 + e1)


def _head(p0, p1, bias, lin_W, lin_b):
    return pl.pallas_call(
        _head_body,
        in_specs=[
            pl.BlockSpec(memory_space=pltpu.VMEM),
            pl.BlockSpec(memory_space=pltpu.VMEM),
            pl.BlockSpec(memory_space=pltpu.VMEM),
            pl.BlockSpec(memory_space=pltpu.VMEM),
            pl.BlockSpec(memory_space=pltpu.SMEM),
        ],
        out_specs=pl.BlockSpec(memory_space=pltpu.SMEM),
        out_shape=jax.ShapeDtypeStruct((1, 2), jnp.float32),
    )(p0, p1, bias, lin_W, lin_b)


def kernel(node_list, edge_list, edge_att, W_l, b_l, W_r, b_r, att, W_e,
           bias, lin_W, lin_b):
    x = node_list[0].astype(jnp.float32)
    src = edge_list[0, 0].astype(jnp.int32)
    dst = edge_list[0, 1].astype(jnp.int32)
    ea = edge_att[0, :, 0].astype(jnp.float32)

    xl, xr = _proj(x, W_l.T, W_r.T, b_l.reshape(1, FEAT), b_r.reshape(1, FEAT))
    att_b = jnp.broadcast_to(att[:, None], (FEAT, 16))
    we_b = jnp.broadcast_to(W_e[:, :1], (FEAT, 16))
    alpha, wmax = _edge_logits(xl, xr, src, dst, ea, att_b, we_b)
    ex, ssump = _edge_exp(alpha, wmax, dst)
    alphan, outparts = _edge_norm(ssump, ex, src, dst, xl)
    pred = _head(outparts[0], outparts[1], bias.reshape(1, FEAT), lin_W,
                 lin_b.reshape(1, 2))
    return pred, alphan


# speedup vs baseline: 2.1563x; 1.0910x over previous
import functools

import jax
import jax.numpy as jnp
from jax import lax
from jax.experimental import pallas as pl
from jax.experimental.pallas import tpu as pltpu
from jax.experimental.pallas import tpu_sc as plsc

N_NODES = 10000
N_EDGES = 320000
FEAT = 128
NC = 2
NS = 16
NW = NC * NS
EPW = N_EDGES // NW
CH = 80
NCHUNK = EPW // CH
NG = CH // 16

_mesh = plsc.VectorSubcoreMesh(core_axis_name="c", subcore_axis_name="s")


def _proj_body(x_ref, wlT_ref, wrT_ref, bl_ref, br_ref, xl_ref, xr_ref):
    x = x_ref[...]
    xl_ref[...] = jnp.dot(x, wlT_ref[...], preferred_element_type=jnp.float32) + bl_ref[...]
    xr_ref[...] = jnp.dot(x, wrT_ref[...], preferred_element_type=jnp.float32) + br_ref[...]


def _proj(x, wlT, wrT, bl, br):
    blk = 1000
    return pl.pallas_call(
        _proj_body,
        grid=(N_NODES // blk,),
        in_specs=[
            pl.BlockSpec((blk, FEAT), lambda i: (i, 0)),
            pl.BlockSpec((FEAT, FEAT), lambda i: (0, 0)),
            pl.BlockSpec((FEAT, FEAT), lambda i: (0, 0)),
            pl.BlockSpec((1, FEAT), lambda i: (0, 0)),
            pl.BlockSpec((1, FEAT), lambda i: (0, 0)),
        ],
        out_specs=[
            pl.BlockSpec((blk, FEAT), lambda i: (i, 0)),
            pl.BlockSpec((blk, FEAT), lambda i: (i, 0)),
        ],
        out_shape=[
            jax.ShapeDtypeStruct((N_NODES, FEAT), jnp.float32),
            jax.ShapeDtypeStruct((N_NODES, FEAT), jnp.float32),
        ],
    )(x, wlT, wrT, bl, br)


@functools.partial(
    pl.kernel,
    out_type=(
        jax.ShapeDtypeStruct((N_EDGES,), jnp.float32),
        jax.ShapeDtypeStruct((NW, 16), jnp.float32),
    ),
    mesh=_mesh,
    compiler_params=pltpu.CompilerParams(needs_layout_passes=False),
    scratch_types=[
        pltpu.VMEM((EPW,), jnp.int32),
        pltpu.VMEM((EPW,), jnp.int32),
        pltpu.VMEM((EPW,), jnp.float32),
        pltpu.VMEM((EPW,), jnp.float32),
        pltpu.VMEM((CH, FEAT), jnp.float32),
        pltpu.VMEM((CH, FEAT), jnp.float32),
        pltpu.VMEM((CH, FEAT), jnp.float32),
        pltpu.VMEM((CH, FEAT), jnp.float32),
        pltpu.VMEM((FEAT, 16), jnp.float32),
        pltpu.VMEM((FEAT, 16), jnp.float32),
        pltpu.VMEM((16,), jnp.float32),
        pltpu.SemaphoreType.DMA,
        pltpu.SemaphoreType.DMA,
    ],
)
def _edge_logits(xl_hbm, xr_hbm, src_hbm, dst_hbm, ea_hbm, att_hbm, we_hbm,
                 alpha_hbm, wmax_hbm,
                 srcall, dstall, eaall, alphas, xlr0, xlr1, xrr0, xrr1,
                 attv, wev, maxbuf, semg0, semg1):
    wid = lax.axis_index("s") * NC + lax.axis_index("c")
    base = wid * EPW
    pltpu.sync_copy(att_hbm, attv)
    pltpu.sync_copy(we_hbm, wev)
    pltpu.sync_copy(src_hbm.at[pl.ds(base, EPW)], srcall)
    pltpu.sync_copy(dst_hbm.at[pl.ds(base, EPW)], dstall)
    pltpu.sync_copy(ea_hbm.at[pl.ds(base, EPW)], eaall)
    rows = [lax.iota(jnp.int32, 16) + (g * 16) for g in range(NG)]
    xbufs = ((xlr0, xrr0, semg0), (xlr1, xrr1, semg1))

    def issue(c, buf):
        xlb, xrb, sem = buf
        sl = pl.ds(c * CH, CH)
        pltpu.async_copy(xl_hbm.at[srcall.at[sl]], xlb, sem)
        pltpu.async_copy(xr_hbm.at[dstall.at[sl]], xrb, sem)

    def wait(buf):
        xlb, xrb, sem = buf
        pltpu.make_async_copy(xl_hbm.at[pl.ds(0, CH)], xlb, sem).wait()
        pltpu.make_async_copy(xr_hbm.at[pl.ds(0, CH)], xrb, sem).wait()

    def compute(c, buf, carry_max):
        xlb, xrb, _ = buf
        eags = [eaall[pl.ds(c * CH + g * 16, 16)] for g in range(NG)]

        def feat_body(ci, accs):
            col = jnp.full((16,), ci, jnp.int32)
            a_c = attv[ci]
            w_c = wev[ci]
            outs = []
            for g in range(NG):
                xl16 = plsc.load_gather(xlb, [rows[g], col])
                xr16 = plsc.load_gather(xrb, [rows[g], col])
                m = xl16 + xr16 + w_c * eags[g]
                m = jnp.maximum(m, 0.2 * m)
                outs.append(accs[g] + a_c * m)
            return tuple(outs)

        accs = lax.fori_loop(0, FEAT, feat_body,
                             tuple(jnp.zeros((16,), jnp.float32) for _ in range(NG)))
        newmax = carry_max
        for g in range(NG):
            alphas[pl.ds(c * CH + g * 16, 16)] = accs[g]
            newmax = jnp.maximum(newmax, accs[g])
        return newmax

    issue(0, xbufs[0])

    def pair_body(i, carry_max):
        a = 2 * i
        issue(a + 1, xbufs[1])
        wait(xbufs[0])
        carry_max = compute(a, xbufs[0], carry_max)
        issue(a + 2, xbufs[0])
        wait(xbufs[1])
        return compute(a + 1, xbufs[1], carry_max)

    mx = lax.fori_loop(0, (NCHUNK - 1) // 2, pair_body,
                       jnp.full((16,), -3.0e38, jnp.float32))
    wait(xbufs[0])
    mx = compute(NCHUNK - 1, xbufs[0], mx)
    maxbuf[...] = mx
    pltpu.sync_copy(alphas, alpha_hbm.at[pl.ds(base, EPW)])
    pltpu.sync_copy(maxbuf, wmax_hbm.at[wid])


@functools.partial(
    pl.kernel,
    out_type=(
        jax.ShapeDtypeStruct((N_EDGES,), jnp.float32),
        jax.ShapeDtypeStruct((NW * N_NODES,), jnp.float32),
    ),
    mesh=_mesh,
    compiler_params=pltpu.CompilerParams(needs_layout_passes=False),
    scratch_types=[
        pltpu.VMEM((EPW,), jnp.float32),
        pltpu.VMEM((EPW,), jnp.int32),
        pltpu.VMEM((N_NODES,), jnp.float32),
        pltpu.VMEM((NW, 16), jnp.float32),
    ],
)
def _edge_exp(alpha_hbm, wmax_hbm, dst_hbm,
              ex_hbm, ssump_hbm,
              alphav, dstv, ssuml, wmaxv):
    wid = lax.axis_index("s") * NC + lax.axis_index("c")
    base = wid * EPW
    pltpu.sync_copy(wmax_hbm, wmaxv)
    m = wmaxv[0]
    for i in range(1, NW):
        m = jnp.maximum(m, wmaxv[i])
    g_shift = jnp.max(m)
    pltpu.sync_copy(alpha_hbm.at[pl.ds(base, EPW)], alphav)
    pltpu.sync_copy(dst_hbm.at[pl.ds(base, EPW)], dstv)

    def zero_body(i, _):
        ssuml[pl.ds(i * 16, 16)] = jnp.zeros((16,), jnp.float32)
        return 0

    lax.fori_loop(0, N_NODES // 16, zero_body, 0)

    def body(i, _):
        a = alphav[pl.ds(i * 16, 16)]
        e = jnp.exp(a - g_shift)
        alphav[pl.ds(i * 16, 16)] = e
        d = dstv[pl.ds(i * 16, 16)]
        plsc.addupdate_scatter(ssuml, [d], e)
        return 0

    lax.fori_loop(0, EPW // 16, body, 0)
    pltpu.sync_copy(alphav, ex_hbm.at[pl.ds(base, EPW)])
    pltpu.sync_copy(ssuml, ssump_hbm.at[pl.ds(wid * N_NODES, N_NODES)])


@functools.partial(
    pl.kernel,
    out_type=(
        jax.ShapeDtypeStruct((N_EDGES,), jnp.float32),
        jax.ShapeDtypeStruct((NC, N_NODES, FEAT), jnp.float32),
    ),
    mesh=_mesh,
    compiler_params=pltpu.CompilerParams(needs_layout_passes=False),
    scratch_types=[
        pltpu.VMEM((N_NODES,), jnp.float32),
        pltpu.VMEM((N_NODES,), jnp.float32),
        pltpu.VMEM((CH,), jnp.int32),
        pltpu.VMEM((1, CH), jnp.int32),
        pltpu.VMEM((CH,), jnp.float32),
        pltpu.VMEM((CH,), jnp.float32),
        pltpu.VMEM((CH, FEAT), jnp.float32),
        pltpu.VMEM((CH, FEAT), jnp.float32),
        pltpu.VMEM_SHARED((N_NODES, FEAT), jnp.float32),
        pltpu.SemaphoreType.DMA,
    ],
)
def _edge_norm(ssump_hbm, ex_hbm, src_hbm, dst_hbm, xl_hbm,
               alphan_hbm, outp_hbm,
               ssumv, tmpv, srcv, dstv, exv, anbuf, xlr, zbuf,
               out_sh, sem):
    cid = lax.axis_index("c")
    sid = lax.axis_index("s")
    wid = sid * NC + cid
    base = wid * EPW
    rows = [lax.iota(jnp.int32, 16) + (g * 16) for g in range(NG)]

    pltpu.sync_copy(ssump_hbm.at[pl.ds(0, N_NODES)], ssumv)
    for r in range(1, NW):
        pltpu.sync_copy(ssump_hbm.at[pl.ds(r * N_NODES, N_NODES)], tmpv)

        def add_body(j, _):
            ssumv[pl.ds(j * 16, 16)] = (ssumv[pl.ds(j * 16, 16)]
                                        + tmpv[pl.ds(j * 16, 16)])
            return 0

        lax.fori_loop(0, N_NODES // 16, add_body, 0)

    @pl.when(sid < 10)
    def _():
        def zero_body(i, _):
            r = i // (FEAT // 16)
            c = i % (FEAT // 16)
            zbuf[r, pl.ds(c * 16, 16)] = jnp.zeros((16,), jnp.float32)
            return 0

        lax.fori_loop(0, CH * (FEAT // 16), zero_body, 0)
        for t in range(12):
            pltpu.sync_copy(zbuf, out_sh.at[pl.ds(sid * 1000 + t * CH, CH)])
        pltpu.sync_copy(zbuf.at[pl.ds(0, 40)],
                        out_sh.at[pl.ds(sid * 1000 + 960, 40)])

    plsc.subcore_barrier()

    def chunk_body(k, _):
        off = base + k * CH
        pltpu.sync_copy(src_hbm.at[pl.ds(off, CH)], srcv)
        pltpu.sync_copy(dst_hbm.at[pl.ds(off, CH)], dstv.at[0])
        pltpu.sync_copy(ex_hbm.at[pl.ds(off, CH)], exv)
        pltpu.async_copy(xl_hbm.at[srcv], xlr, sem).wait()
        angs = []
        for g in range(NG):
            d16 = dstv[0, pl.ds(g * 16, 16)]
            s16 = plsc.load_gather(ssumv, [d16])
            an = exv[pl.ds(g * 16, 16)] / (s16 + 1e-16)
            anbuf[pl.ds(g * 16, 16)] = an
            angs.append(an)

        def feat_body(ci, _):
            col = jnp.full((16,), ci, jnp.int32)
            for g in range(NG):
                v = plsc.load_gather(xlr, [rows[g], col])
                plsc.store_scatter(xlr, [rows[g], col], v * angs[g])
            return 0

        lax.fori_loop(0, FEAT, feat_body, 0)
        pltpu.sync_copy(anbuf, alphan_hbm.at[pl.ds(off, CH)])
        pltpu.sync_copy(xlr, out_sh.at[dstv.at[0]], add=True)
        return 0

    lax.fori_loop(0, NCHUNK, chunk_body, 0)
    plsc.subcore_barrier()

    @pl.when(sid < 10)
    def _():
        pltpu.sync_copy(out_sh.at[pl.ds(sid * 1000, 1000)],
                        outp_hbm.at[cid].at[pl.ds(sid * 1000, 1000)])


def _head_body(p0_ref, p1_ref, b_ref, lw_ref, lb_ref, out_ref):
    h = jnp.maximum(p0_ref[...] + p1_ref[...] + b_ref[...], 0.0)
    pooled = jnp.sum(h, axis=0) * (1.0 / N_NODES)
    s0 = jnp.sum(lw_ref[0, :] * pooled) + lb_ref[0, 0]
    s1 = jnp.sum(lw_ref[1, :] * pooled) + lb_ref[0, 1]
    m = jnp.maximum(s0, s1)
    e0 = jnp.exp(s0 - m)
    e1 = jnp.exp(s1 - m)
    out_ref[0, 0] = e0 / (e0 + e1)
    out_ref[0, 1] = e1 / (e0 + e1)


def _head(p0, p1, bias, lin_W, lin_b):
    return pl.pallas_call(
        _head_body,
        in_specs=[
            pl.BlockSpec(memory_space=pltpu.VMEM),
            pl.BlockSpec(memory_space=pltpu.VMEM),
            pl.BlockSpec(memory_space=pltpu.VMEM),
            pl.BlockSpec(memory_space=pltpu.VMEM),
            pl.BlockSpec(memory_space=pltpu.SMEM),
        ],
        out_specs=pl.BlockSpec(memory_space=pltpu.SMEM),
        out_shape=jax.ShapeDtypeStruct((1, 2), jnp.float32),
    )(p0, p1, bias, lin_W, lin_b)


def kernel(node_list, edge_list, edge_att, W_l, b_l, W_r, b_r, att, W_e,
           bias, lin_W, lin_b):
    x = node_list[0].astype(jnp.float32)
    src = edge_list[0, 0].astype(jnp.int32)
    dst = edge_list[0, 1].astype(jnp.int32)
    ea = edge_att[0, :, 0].astype(jnp.float32)

    xl, xr = _proj(x, W_l.T, W_r.T, b_l.reshape(1, FEAT), b_r.reshape(1, FEAT))
    att_b = jnp.broadcast_to(att[:, None], (FEAT, 16))
    we_b = jnp.broadcast_to(W_e[:, :1], (FEAT, 16))
    alpha, wmax = _edge_logits(xl, xr, src, dst, ea, att_b, we_b)
    ex, ssump = _edge_exp(alpha, wmax, dst)
    alphan, outparts = _edge_norm(ssump, ex, src, dst, xl)
    pred = _head(outparts[0], outparts[1], bias.reshape(1, FEAT), lin_W,
                 lin_b.reshape(1, 2))
    return pred, alphan


# speedup vs baseline: 13.4736x; 6.2485x over previous
import functools

import jax
import jax.numpy as jnp
from jax import lax
from jax.experimental import pallas as pl
from jax.experimental.pallas import tpu as pltpu
from jax.experimental.pallas import tpu_sc as plsc

N_NODES = 10000
N_EDGES = 320000
FEAT = 128
NC = 2
NS = 16
NW = NC * NS
EPW = N_EDGES // NW
CH = 80
NCHUNK = EPW // CH
NG = CH // 16

_mesh = plsc.VectorSubcoreMesh(core_axis_name="c", subcore_axis_name="s")


def _proj_body(x_ref, wlT_ref, wrT_ref, bl_ref, br_ref, xl_ref, xr_ref):
    x = x_ref[...]
    xl_ref[...] = jnp.dot(x, wlT_ref[...], preferred_element_type=jnp.float32) + bl_ref[...]
    xr_ref[...] = jnp.dot(x, wrT_ref[...], preferred_element_type=jnp.float32) + br_ref[...]


def _proj(x, wlT, wrT, bl, br):
    blk = 1000
    return pl.pallas_call(
        _proj_body,
        grid=(N_NODES // blk,),
        in_specs=[
            pl.BlockSpec((blk, FEAT), lambda i: (i, 0)),
            pl.BlockSpec((FEAT, FEAT), lambda i: (0, 0)),
            pl.BlockSpec((FEAT, FEAT), lambda i: (0, 0)),
            pl.BlockSpec((1, FEAT), lambda i: (0, 0)),
            pl.BlockSpec((1, FEAT), lambda i: (0, 0)),
        ],
        out_specs=[
            pl.BlockSpec((blk, FEAT), lambda i: (i, 0)),
            pl.BlockSpec((blk, FEAT), lambda i: (i, 0)),
        ],
        out_shape=[
            jax.ShapeDtypeStruct((N_NODES, FEAT), jnp.float32),
            jax.ShapeDtypeStruct((N_NODES, FEAT), jnp.float32),
        ],
    )(x, wlT, wrT, bl, br)


@functools.partial(
    pl.kernel,
    out_type=(
        jax.ShapeDtypeStruct((N_EDGES,), jnp.float32),
        jax.ShapeDtypeStruct((NW, 16), jnp.float32),
    ),
    mesh=_mesh,
    compiler_params=pltpu.CompilerParams(needs_layout_passes=False),
    scratch_types=[
        pltpu.VMEM((EPW,), jnp.int32),
        pltpu.VMEM((EPW,), jnp.int32),
        pltpu.VMEM((EPW,), jnp.float32),
        pltpu.VMEM((EPW,), jnp.float32),
        pltpu.VMEM((CH, FEAT), jnp.float32),
        pltpu.VMEM((CH, FEAT), jnp.float32),
        pltpu.VMEM((CH, FEAT), jnp.float32),
        pltpu.VMEM((CH, FEAT), jnp.float32),
        pltpu.VMEM((FEAT,), jnp.float32),
        pltpu.VMEM((FEAT,), jnp.float32),
        pltpu.VMEM((16,), jnp.float32),
        pltpu.SemaphoreType.DMA,
        pltpu.SemaphoreType.DMA,
    ],
)
def _edge_logits(xl_hbm, xr_hbm, src_hbm, dst_hbm, ea_hbm, att_hbm, we_hbm,
                 alpha_hbm, wmax_hbm,
                 srcall, dstall, eaall, alphas, xlr0, xlr1, xrr0, xrr1,
                 attv, wev, maxbuf, semg0, semg1):
    wid = lax.axis_index("s") * NC + lax.axis_index("c")
    base = wid * EPW
    pltpu.sync_copy(att_hbm, attv)
    pltpu.sync_copy(we_hbm, wev)
    pltpu.sync_copy(src_hbm.at[pl.ds(base, EPW)], srcall)
    pltpu.sync_copy(dst_hbm.at[pl.ds(base, EPW)], dstall)
    pltpu.sync_copy(ea_hbm.at[pl.ds(base, EPW)], eaall)
    rows = [lax.iota(jnp.int32, 16) + (g * 16) for g in range(NG)]
    xbufs = ((xlr0, xrr0, semg0), (xlr1, xrr1, semg1))

    def issue(c, buf):
        xlb, xrb, sem = buf
        sl = pl.ds(c * CH, CH)
        pltpu.async_copy(xl_hbm.at[srcall.at[sl]], xlb, sem)
        pltpu.async_copy(xr_hbm.at[dstall.at[sl]], xrb, sem)

    def wait(buf):
        xlb, xrb, sem = buf
        pltpu.make_async_copy(xl_hbm.at[pl.ds(0, CH)], xlb, sem).wait()
        pltpu.make_async_copy(xr_hbm.at[pl.ds(0, CH)], xrb, sem).wait()

    att16s = [attv[pl.ds(cb * 16, 16)] for cb in range(FEAT // 16)]
    we16s = [wev[pl.ds(cb * 16, 16)] for cb in range(FEAT // 16)]
    lane = lax.iota(jnp.int32, 16)
    perms = [lane ^ sh for sh in (1, 2, 4, 8)]

    def compute(c, buf, carry_max):
        xlb, xrb, _ = buf
        newmax = carry_max
        for g in range(NG):
            eag = eaall[pl.ds(c * CH + g * 16, 16)]

            def edge_body(u, res):
                e = g * 16 + u
                uidx = jnp.full((16,), u, jnp.int32)
                ea_u = eag.at[uidx].get(mode="promise_in_bounds")
                acc = jnp.zeros((16,), jnp.float32)
                for cb in range(FEAT // 16):
                    xl16 = xlb[e, pl.ds(cb * 16, 16)]
                    xr16 = xrb[e, pl.ds(cb * 16, 16)]
                    m = xl16 + xr16 + we16s[cb] * ea_u
                    m = jnp.maximum(m, 0.2 * m)
                    acc = acc + att16s[cb] * m
                v = acc
                for pidx in perms:
                    v = v + v.at[pidx].get(mode="promise_in_bounds")
                return jnp.where(lane == uidx, v, res)

            res = lax.fori_loop(0, 16, edge_body,
                                jnp.zeros((16,), jnp.float32), unroll=8)
            alphas[pl.ds(c * CH + g * 16, 16)] = res
            newmax = jnp.maximum(newmax, res)
        return newmax

    issue(0, xbufs[0])

    def pair_body(i, carry_max):
        a = 2 * i
        issue(a + 1, xbufs[1])
        wait(xbufs[0])
        carry_max = compute(a, xbufs[0], carry_max)
        issue(a + 2, xbufs[0])
        wait(xbufs[1])
        return compute(a + 1, xbufs[1], carry_max)

    mx = lax.fori_loop(0, (NCHUNK - 1) // 2, pair_body,
                       jnp.full((16,), -3.0e38, jnp.float32))
    wait(xbufs[0])
    mx = compute(NCHUNK - 1, xbufs[0], mx)
    maxbuf[...] = mx
    pltpu.sync_copy(alphas, alpha_hbm.at[pl.ds(base, EPW)])
    pltpu.sync_copy(maxbuf, wmax_hbm.at[wid])


@functools.partial(
    pl.kernel,
    out_type=(
        jax.ShapeDtypeStruct((N_EDGES,), jnp.float32),
        jax.ShapeDtypeStruct((NW * N_NODES,), jnp.float32),
    ),
    mesh=_mesh,
    compiler_params=pltpu.CompilerParams(needs_layout_passes=False),
    scratch_types=[
        pltpu.VMEM((EPW,), jnp.float32),
        pltpu.VMEM((EPW,), jnp.int32),
        pltpu.VMEM((N_NODES,), jnp.float32),
        pltpu.VMEM((NW, 16), jnp.float32),
    ],
)
def _edge_exp(alpha_hbm, wmax_hbm, dst_hbm,
              ex_hbm, ssump_hbm,
              alphav, dstv, ssuml, wmaxv):
    wid = lax.axis_index("s") * NC + lax.axis_index("c")
    base = wid * EPW
    pltpu.sync_copy(wmax_hbm, wmaxv)
    m = wmaxv[0]
    for i in range(1, NW):
        m = jnp.maximum(m, wmaxv[i])
    g_shift = jnp.max(m)
    pltpu.sync_copy(alpha_hbm.at[pl.ds(base, EPW)], alphav)
    pltpu.sync_copy(dst_hbm.at[pl.ds(base, EPW)], dstv)

    def zero_body(i, _):
        ssuml[pl.ds(i * 16, 16)] = jnp.zeros((16,), jnp.float32)
        return 0

    lax.fori_loop(0, N_NODES // 16, zero_body, 0)

    def body(i, _):
        a = alphav[pl.ds(i * 16, 16)]
        e = jnp.exp(a - g_shift)
        alphav[pl.ds(i * 16, 16)] = e
        d = dstv[pl.ds(i * 16, 16)]
        plsc.addupdate_scatter(ssuml, [d], e)
        return 0

    lax.fori_loop(0, EPW // 16, body, 0)
    pltpu.sync_copy(alphav, ex_hbm.at[pl.ds(base, EPW)])
    pltpu.sync_copy(ssuml, ssump_hbm.at[pl.ds(wid * N_NODES, N_NODES)])


@functools.partial(
    pl.kernel,
    out_type=(
        jax.ShapeDtypeStruct((N_EDGES,), jnp.float32),
        jax.ShapeDtypeStruct((NC, N_NODES, FEAT), jnp.float32),
    ),
    mesh=_mesh,
    compiler_params=pltpu.CompilerParams(needs_layout_passes=False),
    scratch_types=[
        pltpu.VMEM((N_NODES,), jnp.float32),
        pltpu.VMEM((2000,), jnp.float32),
        pltpu.VMEM((2000,), jnp.float32),
        pltpu.VMEM((2000,), jnp.float32),
        pltpu.VMEM((CH,), jnp.int32),
        pltpu.VMEM((CH,), jnp.int32),
        pltpu.VMEM((1, CH), jnp.int32),
        pltpu.VMEM((1, CH), jnp.int32),
        pltpu.VMEM((1, CH), jnp.int32),
        pltpu.VMEM((1, CH), jnp.int32),
        pltpu.VMEM((CH,), jnp.float32),
        pltpu.VMEM((CH,), jnp.float32),
        pltpu.VMEM((CH,), jnp.float32),
        pltpu.VMEM((CH,), jnp.float32),
        pltpu.VMEM((CH, FEAT), jnp.float32),
        pltpu.VMEM((CH, FEAT), jnp.float32),
        pltpu.VMEM((40, FEAT), jnp.float32),
        pltpu.VMEM_SHARED((N_NODES,), jnp.float32),
        pltpu.VMEM_SHARED((N_NODES, FEAT), jnp.float32),
        pltpu.SemaphoreType.DMA,
        pltpu.SemaphoreType.DMA,
        pltpu.SemaphoreType.DMA,
        pltpu.SemaphoreType.DMA,
        pltpu.SemaphoreType.DMA,
        pltpu.SemaphoreType.DMA,
        pltpu.SemaphoreType.DMA,
        pltpu.SemaphoreType.DMA,
        pltpu.SemaphoreType.DMA,
        pltpu.SemaphoreType.DMA,
    ],
)
def _edge_norm(ssump_hbm, ex_hbm, src_hbm, dst_hbm, xl_hbm,
               alphan_hbm, outp_hbm,
               ssumv, red0, red1, ssfold, srcb0, srcb1, dstb0, dstb1,
               dstS0, dstS1, exb0, exb1, anb0, anb1, xlr0, xlr1, zbuf,
               ssum_sh, out_sh, semr0, semr1,
               semi0, semi1, semg0, semg1, sems0, sems1, sema0, sema1):
    cid = lax.axis_index("c")
    sid = lax.axis_index("s")
    wid = sid * NC + cid
    base = wid * EPW

    reds = ((red0, semr0), (red1, semr1))

    def rissue(r, rb):
        buf, sem = rb
        pltpu.async_copy(
            ssump_hbm.at[pl.ds(r * N_NODES + sid * 2000, 2000)], buf, sem)

    def rwait(rb):
        buf, sem = rb
        pltpu.make_async_copy(ssump_hbm.at[pl.ds(0, 2000)], buf, sem).wait()

    @pl.when(sid < 5)
    def _():
        def zf(j, _):
            ssfold[pl.ds(j * 16, 16)] = jnp.zeros((16,), jnp.float32)
            return 0

        lax.fori_loop(0, 125, zf, 0)
        rissue(0, reds[0])
        rissue(1, reds[1])
        for r in range(NW):
            rb = reds[r % 2]
            rwait(rb)

            def fold_body(j, _):
                ssfold[pl.ds(j * 16, 16)] = (ssfold[pl.ds(j * 16, 16)]
                                             + rb[0][pl.ds(j * 16, 16)])
                return 0

            lax.fori_loop(0, 125, fold_body, 0, unroll=4)
            if r + 2 < NW:
                rissue(r + 2, rb)
        pltpu.sync_copy(ssfold, ssum_sh.at[pl.ds(sid * 2000, 2000)])

    @pl.when(sid < 10)
    def _():
        def zero_body(i, _):
            r = i // (FEAT // 16)
            c = i % (FEAT // 16)
            zbuf[r, pl.ds(c * 16, 16)] = jnp.zeros((16,), jnp.float32)
            return 0

        lax.fori_loop(0, 40 * (FEAT // 16), zero_body, 0)
        for t in range(25):
            pltpu.sync_copy(zbuf, out_sh.at[pl.ds(sid * 1000 + t * 40, 40)])

    plsc.subcore_barrier()
    pltpu.sync_copy(ssum_sh, ssumv)

    bufs = ((srcb0, dstb0, dstS0, exb0, anb0, xlr0, semi0, semg0, sems0, sema0),
            (srcb1, dstb1, dstS1, exb1, anb1, xlr1, semi1, semg1, sems1, sema1))

    def issue_idx(c, b):
        srcb, dstb, _, exb, _, _, semi, _, _, _ = b
        off = base + c * CH
        pltpu.async_copy(src_hbm.at[pl.ds(off, CH)], srcb, semi)
        pltpu.async_copy(dst_hbm.at[pl.ds(off, CH)], dstb.at[0], semi)
        pltpu.async_copy(ex_hbm.at[pl.ds(off, CH)], exb, semi)

    def wait_idx(b):
        srcb, dstb, _, exb, _, _, semi, _, _, _ = b
        pltpu.make_async_copy(src_hbm.at[pl.ds(0, CH)], srcb, semi).wait()
        pltpu.make_async_copy(dst_hbm.at[pl.ds(0, CH)], dstb.at[0], semi).wait()
        pltpu.make_async_copy(ex_hbm.at[pl.ds(0, CH)], exb, semi).wait()

    def issue_g(b):
        srcb, _, _, _, _, xlrb, _, semg, _, _ = b
        pltpu.async_copy(xl_hbm.at[srcb], xlrb, semg)

    def wait_g(b):
        _, _, _, _, _, xlrb, _, semg, _, _ = b
        pltpu.make_async_copy(xl_hbm.at[pl.ds(0, CH)], xlrb, semg).wait()

    def compute(b):
        _, dstb, dstS, exb, anb, xlrb, _, _, _, _ = b
        angs = []
        for g in range(NG):
            sl = pl.ds(g * 16, 16)
            d16 = dstb[0, sl]
            s16 = plsc.load_gather(ssumv, [d16])
            an = exb[sl] / (s16 + 1e-16)
            anb[sl] = an
            dstS[0, sl] = d16
            angs.append(an)
        for g in range(NG):
            def edge_body(u, _2):
                e = g * 16 + u
                uidx = jnp.full((16,), u, jnp.int32)
                a_u = angs[g].at[uidx].get(mode="promise_in_bounds")
                for cb in range(FEAT // 16):
                    csl = pl.ds(cb * 16, 16)
                    xlrb[e, csl] = xlrb[e, csl] * a_u
                return 0

            lax.fori_loop(0, 16, edge_body, 0, unroll=4)

    def issue_out(c, b):
        _, _, dstS, _, anb, xlrb, _, _, semsc, sema = b
        off = base + c * CH
        pltpu.async_copy(anb, alphan_hbm.at[pl.ds(off, CH)], sema)
        pltpu.async_copy(xlrb, out_sh.at[dstS.at[0]], semsc, add=True)

    def wait_out(b):
        _, _, dstS, _, anb, xlrb, _, _, semsc, sema = b
        pltpu.make_async_copy(anb, alphan_hbm.at[pl.ds(base, CH)], sema).wait()
        pltpu.make_async_copy(xlrb, out_sh.at[dstS.at[0]], semsc).wait()

    b0, b1 = bufs
    issue_idx(0, b0)
    wait_idx(b0)
    issue_g(b0)
    issue_idx(1, b1)

    def pair_body(i, _):
        a = 2 * i

        @pl.when(i > 0)
        def _():
            wait_out(b1)

        wait_g(b0)
        compute(b0)
        issue_out(a, b0)
        issue_idx(a + 2, b0)
        wait_idx(b1)
        issue_g(b1)
        wait_g(b1)
        compute(b1)
        issue_out(a + 1, b1)

        @pl.when(i < (NCHUNK - 1) // 2 - 1)
        def _():
            issue_idx(a + 3, b1)

        wait_idx(b0)
        wait_out(b0)
        issue_g(b0)
        return 0

    lax.fori_loop(0, (NCHUNK - 1) // 2, pair_body, 0)
    wait_out(b1)
    wait_g(b0)
    compute(b0)
    issue_out(NCHUNK - 1, b0)
    wait_out(b0)
    plsc.subcore_barrier()

    @pl.when(sid < 10)
    def _():
        pltpu.sync_copy(out_sh.at[pl.ds(sid * 1000, 1000)],
                        outp_hbm.at[cid].at[pl.ds(sid * 1000, 1000)])


def _head_body(p0_ref, p1_ref, b_ref, lw_ref, lb_ref, out_ref):
    h = jnp.maximum(p0_ref[...] + p1_ref[...] + b_ref[...], 0.0)
    pooled = jnp.sum(h, axis=0) * (1.0 / N_NODES)
    s0 = jnp.sum(lw_ref[0, :] * pooled) + lb_ref[0, 0]
    s1 = jnp.sum(lw_ref[1, :] * pooled) + lb_ref[0, 1]
    m = jnp.maximum(s0, s1)
    e0 = jnp.exp(s0 - m)
    e1 = jnp.exp(s1 - m)
    out_ref[0, 0] = e0 / (e0 + e1)
    out_ref[0, 1] = e1 / (e0 + e1)


def _head(p0, p1, bias, lin_W, lin_b):
    return pl.pallas_call(
        _head_body,
        in_specs=[
            pl.BlockSpec(memory_space=pltpu.VMEM),
            pl.BlockSpec(memory_space=pltpu.VMEM),
            pl.BlockSpec(memory_space=pltpu.VMEM),
            pl.BlockSpec(memory_space=pltpu.VMEM),
            pl.BlockSpec(memory_space=pltpu.SMEM),
        ],
        out_specs=pl.BlockSpec(memory_space=pltpu.SMEM),
        out_shape=jax.ShapeDtypeStruct((1, 2), jnp.float32),
    )(p0, p1, bias, lin_W, lin_b)


def kernel(node_list, edge_list, edge_att, W_l, b_l, W_r, b_r, att, W_e,
           bias, lin_W, lin_b):
    x = node_list[0].astype(jnp.float32)
    src = edge_list[0, 0].astype(jnp.int32)
    dst = edge_list[0, 1].astype(jnp.int32)
    ea = edge_att[0, :, 0].astype(jnp.float32)

    xl, xr = _proj(x, W_l.T, W_r.T, b_l.reshape(1, FEAT), b_r.reshape(1, FEAT))
    alpha, wmax = _edge_logits(xl, xr, src, dst, ea, att, W_e[:, 0])
    ex, ssump = _edge_exp(alpha, wmax, dst)
    alphan, outparts = _edge_norm(ssump, ex, src, dst, xl)
    pred = _head(outparts[0], outparts[1], bias.reshape(1, FEAT), lin_W,
                 lin_b.reshape(1, 2))
    return pred, alphan
